# baseline (speedup 1.0000x reference)
import functools

import jax
import jax.numpy as jnp
from jax import lax
from jax.experimental import pallas as pl
from jax.experimental.pallas import tpu as pltpu
from jax.experimental.pallas import tpu_sc as plsc

NC = 2
NS = 16
NW = NC * NS
NCHUNK = 4
CH = 64
DEPTH = 4
NROWBUF = 2 * DEPTH


def _make_embed(CS, S1, D):
    IW = CS // NW
    NB = IW // CH
    K = S1 * NB
    assert CS % NW == 0 and IW % CH == 0
    assert K >= 2 * DEPTH and (K - 2 * DEPTH) % NROWBUF == 0
    mesh = plsc.VectorSubcoreMesh(core_axis_name="c", subcore_axis_name="s")
    sems = [pltpu.SemaphoreType.DMA for _ in range(NROWBUF)]

    @functools.partial(
        pl.kernel,
        out_type=jax.ShapeDtypeStruct((CS, S1, D), jnp.float32),
        mesh=mesh,
        scratch_types=[
            pltpu.VMEM((S1, IW), jnp.int32),
            pltpu.VMEM((NROWBUF, CH, D), jnp.float32),
        ] + sems,
        compiler_params=pltpu.CompilerParams(use_tc_tiling_on_sc=False),
    )
    def embed(xt_hbm, table_hbm, out_hbm, idx_v, rows_v, *sem):
        wid = lax.axis_index("s") * NC + lax.axis_index("c")
        i0 = wid * IW

        pltpu.sync_copy(xt_hbm.at[:, pl.ds(i0, IW)], idx_v)

        def scale(j, carry):
            for k in range(IW // 16):
                idx_v[j, pl.ds(k * 16, 16)] = idx_v[j, pl.ds(k * 16, 16)] * 4
            return carry

        lax.fori_loop(0, S1, scale, 0)

        def idx_slice(r):
            return idx_v.at[r // NB, pl.ds((r % NB) * CH, CH)]

        def out_slice(r):
            return out_hbm.at[pl.ds(i0 + (r % NB) * CH, CH), r // NB]

        def start_gather(r, b):
            pltpu.async_copy(table_hbm.at[idx_slice(r)], rows_v.at[b], sem[b])

        def wait_gather(r, b):
            pltpu.make_async_copy(
                table_hbm.at[idx_slice(r)], rows_v.at[b], sem[b]
            ).wait()

        def start_write(r, b):
            pltpu.async_copy(rows_v.at[b], out_slice(r), sem[b])

        def wait_write(r, b):
            pltpu.make_async_copy(rows_v.at[b], out_slice(r), sem[b]).wait()

        for r in range(DEPTH):
            start_gather(r, r)

        for r in range(DEPTH):
            wait_gather(r, r)
            start_write(r, r)
            start_gather(r + DEPTH, r + DEPTH)

        def block(i, carry):
            r0 = DEPTH + i * NROWBUF
            for t in range(NROWBUF):
                r = r0 + t
                bg = (DEPTH + t) % NROWBUF
                bn = t
                wait_gather(r, bg)
                start_write(r, bg)
                wait_write(r - DEPTH, bn)
                start_gather(r + DEPTH, bn)
            return carry

        lax.fori_loop(0, (K - 2 * DEPTH) // NROWBUF, block, 0)

        for r in range(K - DEPTH, K):
            wait_gather(r, r % NROWBUF)
            start_write(r, r % NROWBUF)
        for r in range(K - NROWBUF, K):
            wait_write(r, r % NROWBUF)

    return embed


TCB = 4096


def _tc_fatten(wt, V, D):
    G = (V + TCB - 1) // TCB

    def body(in_ref, out_ref):
        out_ref[:, 0:D] = in_ref[...].T

    return pl.pallas_call(
        body,
        grid=(G,),
        in_specs=[pl.BlockSpec((D, TCB), lambda g: (0, g))],
        out_specs=pl.BlockSpec((TCB, 128), lambda g: (g, 0)),
        out_shape=jax.ShapeDtypeStruct((V, 128), jnp.float32),
    )(wt)


def kernel(x, weight):
    S0, S1 = x.shape
    V, D = weight.shape
    xt = x.astype(jnp.int32).T
    wfat = _tc_fatten(weight.T, V, D).reshape(4 * V, D)
    CS = S0 // NCHUNK
    embed = _make_embed(CS, S1, D)
    parts = [embed(xt[:, c * CS:(c + 1) * CS], wfat) for c in range(NCHUNK)]
    return jnp.concatenate(parts, axis=0)


# speedup vs baseline: 1.0190x; 1.0190x over previous
import functools

import jax
import jax.numpy as jnp
from jax import lax
from jax.experimental import pallas as pl
from jax.experimental.pallas import tpu as pltpu
from jax.experimental.pallas import tpu_sc as plsc

NC = 2
NS = 16
NW = NC * NS
NCHUNK = 8
CH = 64
DEPTH = 2
NROWBUF = 2 * DEPTH


def _make_embed(CS, S1, D):
    IW = CS // NW
    NB = IW // CH
    K = S1 * NB
    assert CS % NW == 0 and IW % CH == 0
    assert K >= 2 * DEPTH and (K - 2 * DEPTH) % NROWBUF == 0
    mesh = plsc.VectorSubcoreMesh(core_axis_name="c", subcore_axis_name="s")
    sems = [pltpu.SemaphoreType.DMA for _ in range(NROWBUF)]

    @functools.partial(
        pl.kernel,
        out_type=jax.ShapeDtypeStruct((CS, S1, D), jnp.float32),
        mesh=mesh,
        scratch_types=[
            pltpu.VMEM((S1, IW), jnp.int32),
            pltpu.VMEM((NROWBUF, CH, D), jnp.float32),
        ] + sems,
        compiler_params=pltpu.CompilerParams(use_tc_tiling_on_sc=False),
    )
    def embed(xt_hbm, table_hbm, out_hbm, idx_v, rows_v, *sem):
        wid = lax.axis_index("s") * NC + lax.axis_index("c")
        i0 = wid * IW

        pltpu.sync_copy(xt_hbm.at[:, pl.ds(i0, IW)], idx_v)

        def scale(j, carry):
            for k in range(IW // 16):
                idx_v[j, pl.ds(k * 16, 16)] = idx_v[j, pl.ds(k * 16, 16)] * 4
            return carry

        lax.fori_loop(0, S1, scale, 0)

        def idx_slice(r):
            return idx_v.at[r // NB, pl.ds((r % NB) * CH, CH)]

        def out_slice(r):
            return out_hbm.at[pl.ds(i0 + (r % NB) * CH, CH), r // NB]

        def start_gather(r, b):
            pltpu.async_copy(table_hbm.at[idx_slice(r)], rows_v.at[b], sem[b])

        def wait_gather(r, b):
            pltpu.make_async_copy(
                table_hbm.at[idx_slice(r)], rows_v.at[b], sem[b]
            ).wait()

        def start_write(r, b):
            pltpu.async_copy(rows_v.at[b], out_slice(r), sem[b])

        def wait_write(r, b):
            pltpu.make_async_copy(rows_v.at[b], out_slice(r), sem[b]).wait()

        for r in range(DEPTH):
            start_gather(r, r)

        for r in range(DEPTH):
            wait_gather(r, r)
            start_write(r, r)
            start_gather(r + DEPTH, r + DEPTH)

        def block(i, carry):
            r0 = DEPTH + i * NROWBUF
            for t in range(NROWBUF):
                r = r0 + t
                bg = (DEPTH + t) % NROWBUF
                bn = t
                wait_gather(r, bg)
                start_write(r, bg)
                wait_write(r - DEPTH, bn)
                start_gather(r + DEPTH, bn)
            return carry

        lax.fori_loop(0, (K - 2 * DEPTH) // NROWBUF, block, 0)

        for r in range(K - DEPTH, K):
            wait_gather(r, r % NROWBUF)
            start_write(r, r % NROWBUF)
        for r in range(K - NROWBUF, K):
            wait_write(r, r % NROWBUF)

    return embed


TCB = 4096


def _tc_fatten(wt, V, D):
    G = (V + TCB - 1) // TCB

    def body(in_ref, out_ref):
        out_ref[:, 0:D] = in_ref[...].T

    return pl.pallas_call(
        body,
        grid=(G,),
        in_specs=[pl.BlockSpec((D, TCB), lambda g: (0, g))],
        out_specs=pl.BlockSpec((TCB, 128), lambda g: (g, 0)),
        out_shape=jax.ShapeDtypeStruct((V, 128), jnp.float32),
    )(wt)


def kernel(x, weight):
    S0, S1 = x.shape
    V, D = weight.shape
    xt = x.astype(jnp.int32).T
    wfat = _tc_fatten(weight.T, V, D).reshape(4 * V, D)
    CS = S0 // NCHUNK
    embed = _make_embed(CS, S1, D)
    parts = [embed(xt[:, c * CS:(c + 1) * CS], wfat) for c in range(NCHUNK)]
    return jnp.concatenate(parts, axis=0)
